# Initial kernel scaffold; baseline (speedup 1.0000x reference)
import jax
import jax.numpy as jnp
from jax.experimental import pallas as pl


def kernel(x):
    raise NotImplementedError("write your pallas kernel here")


# speedup vs baseline: 2.0317x; 2.0317x over previous
import jax
import jax.numpy as jnp
from jax import lax
from jax.experimental import pallas as pl

_ANCHOR_W = (10.0, 16.0, 33.0)
_ANCHOR_H = (13.0, 30.0, 23.0)
_IMG_DIM = 608.0


def _body(x_ref, o_ref, *, in_h, stride):
    i = pl.program_id(0)
    a = i % 3
    hw = in_h * in_h
    v = x_ref[0]

    xy = jax.nn.sigmoid(v[0:2])
    wh = jnp.exp(v[2:4])
    rest = jax.nn.sigmoid(v[4:])

    n = lax.broadcasted_iota(jnp.int32, (1, hw), 1)
    gx = (n % in_h).astype(jnp.float32)
    gy = (n // in_h).astype(jnp.float32)

    row0 = (xy[0:1] + gx) * stride
    row1 = (xy[1:2] + gy) * stride
    aw = jnp.where(a == 0, _ANCHOR_W[0], jnp.where(a == 1, _ANCHOR_W[1], _ANCHOR_W[2]))
    ah = jnp.where(a == 0, _ANCHOR_H[0], jnp.where(a == 1, _ANCHOR_H[1], _ANCHOR_H[2]))
    row2 = wh[0:1] * aw
    row3 = wh[1:2] * ah

    out85 = jnp.concatenate([row0, row1, row2, row3, rest], axis=0)
    o_ref[0] = out85.T


def kernel(x):
    bs, ch, in_h, _ = x.shape
    na = 3
    attrs = ch // na
    hw = in_h * in_h
    stride = _IMG_DIM / in_h

    x3 = x.reshape(bs * na, attrs, hw)
    import functools
    body = functools.partial(_body, in_h=in_h, stride=stride)
    out = pl.pallas_call(
        body,
        grid=(bs * na,),
        in_specs=[pl.BlockSpec((1, attrs, hw), lambda i: (i, 0, 0))],
        out_specs=pl.BlockSpec((1, hw, attrs), lambda i: (i, 0, 0)),
        out_shape=jax.ShapeDtypeStruct((bs * na, hw, attrs), jnp.float32),
    )(x3)
    return out.reshape(bs, na * hw, attrs)


# speedup vs baseline: 2.7669x; 1.3618x over previous
import jax
import jax.numpy as jnp
from jax import lax
from jax.experimental import pallas as pl

_ANCHOR_W = (10.0, 16.0, 33.0)
_ANCHOR_H = (13.0, 30.0, 23.0)
_IMG_DIM = 608.0


def _body(x_ref, o_ref, *, in_h, stride):
    a = pl.program_id(1)
    hw = in_h * in_h
    v = x_ref[0]

    xy = jax.nn.sigmoid(v[0:2])
    wh = jnp.exp(v[2:4])
    rest = jax.nn.sigmoid(v[4:])

    gy = lax.broadcasted_iota(jnp.int32, (1, in_h, in_h), 1).astype(jnp.float32)
    gx = lax.broadcasted_iota(jnp.int32, (1, in_h, in_h), 2).astype(jnp.float32)

    row0 = (xy[0:1] + gx) * stride
    row1 = (xy[1:2] + gy) * stride
    aw = jnp.where(a == 0, _ANCHOR_W[0], jnp.where(a == 1, _ANCHOR_W[1], _ANCHOR_W[2]))
    ah = jnp.where(a == 0, _ANCHOR_H[0], jnp.where(a == 1, _ANCHOR_H[1], _ANCHOR_H[2]))
    row2 = wh[0:1] * aw
    row3 = wh[1:2] * ah

    out85 = jnp.concatenate([row0, row1, row2, row3, rest], axis=0)
    o_ref[0, 0] = out85.reshape(85, hw).T


def kernel(x):
    bs, ch, in_h, _ = x.shape
    na = 3
    attrs = ch // na
    hw = in_h * in_h
    stride = _IMG_DIM / in_h

    import functools
    body = functools.partial(_body, in_h=in_h, stride=stride)
    out = pl.pallas_call(
        body,
        grid=(bs, na),
        in_specs=[pl.BlockSpec((1, attrs, in_h, in_h), lambda b, a: (b, a, 0, 0))],
        out_specs=pl.BlockSpec((1, 1, hw, attrs), lambda b, a: (b, a, 0, 0)),
        out_shape=jax.ShapeDtypeStruct((bs, na, hw, attrs), jnp.float32),
    )(x)
    return out.reshape(bs, na * hw, attrs)


# speedup vs baseline: 4.8117x; 1.7391x over previous
import functools

import jax
import jax.numpy as jnp
from jax import lax
from jax.experimental import pallas as pl

_ANCHOR_W = (10.0, 16.0, 33.0)
_ANCHOR_H = (13.0, 30.0, 23.0)
_IMG_DIM = 608.0


def _body(x0_ref, x1_ref, x2_ref, o_ref, *, bs, in_h, stride):
    cb = pl.program_id(0)
    hw = in_h * in_h
    refs = (x0_ref, x1_ref, x2_ref)

    n = lax.broadcasted_iota(jnp.int32, (1, hw), 1)
    gx = (n % in_h).astype(jnp.float32)
    gy = (n // in_h).astype(jnp.float32)

    for a in range(3):
        sl = pl.ds(a * hw, hw)
        for i in range(5):
            v = refs[a][:, i].reshape(bs, hw)
            o_ref[i, :, sl] = jax.nn.sigmoid(v)

        @pl.when(cb == 0)
        def _(a=a, sl=sl):
            v0 = refs[a][:, 0].reshape(bs, hw)
            v1 = refs[a][:, 1].reshape(bs, hw)
            v2 = refs[a][:, 2].reshape(bs, hw)
            v3 = refs[a][:, 3].reshape(bs, hw)
            o_ref[0, :, sl] = (jax.nn.sigmoid(v0) + gx) * stride
            o_ref[1, :, sl] = (jax.nn.sigmoid(v1) + gy) * stride
            o_ref[2, :, sl] = jnp.exp(v2) * _ANCHOR_W[a]
            o_ref[3, :, sl] = jnp.exp(v3) * _ANCHOR_H[a]


def kernel(x):
    bs, ch, in_h, _ = x.shape
    na = 3
    attrs = ch // na
    hw = in_h * in_h
    stride = _IMG_DIM / in_h
    cgrp = 5
    ngrp = attrs // cgrp

    body = functools.partial(_body, bs=bs, in_h=in_h, stride=stride)

    def in_spec(a):
        return pl.BlockSpec(
            (bs, cgrp, in_h, in_h), lambda cb, a=a: (0, a * ngrp + cb, 0, 0)
        )

    out = pl.pallas_call(
        body,
        grid=(ngrp,),
        in_specs=[in_spec(0), in_spec(1), in_spec(2)],
        out_specs=pl.BlockSpec((cgrp, bs, na * hw), lambda cb: (cb, 0, 0)),
        out_shape=jax.ShapeDtypeStruct((attrs, bs, na * hw), jnp.float32),
    )(x, x, x)
    return out.transpose(1, 2, 0)
